# baseline (speedup 1.0000x reference)
import jax
import jax.numpy as jnp
from jax import lax
from jax.experimental import pallas as pl
from jax.experimental.pallas import tpu as pltpu

EPS = 1e-06
TOP_M = 64
KPAD = 128
N_STEPS = 16
LG_STEPS = 4
SEL_START = 5
SEL_PER_STEP = 8
BIG_I32 = 2 ** 30


def _lane_allreduce(x, op):
    for sh in (1, 2, 4, 8, 16, 32, 64):
        x = op(x, pltpu.roll(x, sh, 1))
    return x


def _onehot(y_ref, base, n):
    yb = y_ref[pl.ds(base, n), :]
    ks = jax.lax.broadcasted_iota(jnp.int32, (n, KPAD), 1)
    return (yb == ks).astype(jnp.float32)


def _fused(f1_ref, f2_ref, lg_ref, y_ref, out_ref,
           cr_ref, cc_ref, s1_ref, q1_ref, s2_ref, q2_ref, p_ref,
           amat_ref, seli_ref, selj_ref, avec_ref):
    step = pl.program_id(0)
    B = y_ref.shape[0]
    bB = f1_ref.shape[0]
    bL = lg_ref.shape[0]
    K = lg_ref.shape[1]
    dn = (((0,), (0,)), ((), ()))

    @pl.when(step == 0)
    def _init():
        cr_ref[...] = jnp.zeros_like(cr_ref)
        cc_ref[...] = jnp.zeros_like(cc_ref)
        s1_ref[...] = jnp.zeros_like(s1_ref)
        q1_ref[...] = jnp.zeros_like(q1_ref)
        s2_ref[...] = jnp.zeros_like(s2_ref)
        q2_ref[...] = jnp.zeros_like(q2_ref)
        p_ref[...] = jnp.zeros_like(p_ref)

    z1 = f1_ref[...]
    z2 = f2_ref[...]
    oh = _onehot(y_ref, step * bB, bB)
    s1_ref[...] += jax.lax.dot_general(oh, z1, dn, preferred_element_type=jnp.float32)
    q1_ref[...] += jax.lax.dot_general(oh, z1 * z1, dn, preferred_element_type=jnp.float32)
    s2_ref[...] += jax.lax.dot_general(oh, z2, dn, preferred_element_type=jnp.float32)
    q2_ref[...] += jax.lax.dot_general(oh, z2 * z2, dn, preferred_element_type=jnp.float32)

    @pl.when(step < LG_STEPS)
    def _logits():
        oh4 = _onehot(y_ref, step * bL, bL)
        cr_ref[...] += jnp.sum(oh4, axis=0, keepdims=True)
        cc_ref[...] += jax.lax.dot_general(
            oh4, jnp.ones((bL, 1), jnp.float32), dn,
            preferred_element_type=jnp.float32)
        lg = lg_ref[...]
        m = jnp.max(lg, axis=1, keepdims=True)
        e = jnp.exp(lg - m)
        p = e / jnp.sum(e, axis=1, keepdims=True)
        p_ref[...] += jax.lax.dot_general(oh4, p, dn,
                                          preferred_element_type=jnp.float32)

    @pl.when(step == LG_STEPS)
    def _build_amat():
        c_row = cr_ref[...]
        c_col = cc_ref[...]
        rinv = 1.0 / jnp.maximum(c_col, 1.0)
        valid_row = (c_row >= 2.0)
        valid_col = (c_col >= 2.0)
        mean_p = p_ref[...] * rinv
        ei = jax.lax.broadcasted_iota(jnp.int32, (K, KPAD), 0)
        ej = jax.lax.broadcasted_iota(jnp.int32, (K, KPAD), 1)
        pad_eye = (ei == ej).astype(jnp.float32)
        mp = jax.lax.dot_general(mean_p, pad_eye, (((1,), (0,)), ((), ())),
                                 preferred_element_type=jnp.float32)
        alpha = 0.5 * (mp + mp.T)
        ri = jax.lax.broadcasted_iota(jnp.int32, (KPAD, KPAD), 0)
        cj = jax.lax.broadcasted_iota(jnp.int32, (KPAD, KPAD), 1)
        keep = jnp.logical_and(cj > ri, jnp.logical_and(valid_col, valid_row))
        amat_ref[...] = jnp.where(keep, alpha, 0.0)

    ri = jax.lax.broadcasted_iota(jnp.int32, (KPAD, KPAD), 0)
    cj = jax.lax.broadcasted_iota(jnp.int32, (KPAD, KPAD), 1)
    fidx = ri * KPAD + cj
    fidxT = cj * KPAD + ri

    for s in range(SEL_START, SEL_START + TOP_M // SEL_PER_STEP):
        @pl.when(step == s)
        def _select(s=s):
            amat = amat_ref[...]
            for it in range(SEL_PER_STEP):
                g = (s - SEL_START) * SEL_PER_STEP + it
                colmax = jnp.max(amat, axis=0, keepdims=True)
                allmax = _lane_allreduce(colmax, jnp.maximum)
                cand = jnp.where(amat == allmax, fidx, BIG_I32)
                colmin = jnp.min(cand, axis=0, keepdims=True)
                allmin = _lane_allreduce(colmin, jnp.minimum)
                msel = fidx == allmin
                jind = jnp.sum(msel.astype(jnp.float32), axis=0, keepdims=True)
                iind = jnp.sum((fidxT == allmin).astype(jnp.float32), axis=0,
                               keepdims=True)
                seli_ref[pl.ds(g, 1), :] = iind
                selj_ref[pl.ds(g, 1), :] = jind
                avec_ref[pl.ds(g, 1), :] = allmax[:, :1]
                amat = jnp.where(msel, -1.0, amat)
            amat_ref[...] = amat

    @pl.when(step == N_STEPS - 1)
    def _finale():
        c_row = cr_ref[...]
        c_col = cc_ref[...]
        rinv = 1.0 / jnp.maximum(c_col, 1.0)
        valid_row = (c_row >= 2.0)
        num_valid = jnp.sum(valid_row.astype(jnp.float32))
        kept = jnp.sum(jnp.where(valid_row, c_row, 0.0))
        layer_valid = jnp.logical_and(num_valid >= 2.0, kept >= 4.0)

        sel_i = seli_ref[...]
        sel_j = selj_ref[...]
        sel_d = sel_i - sel_j
        avec = avec_ref[...]
        dnm = (((1,), (0,)), ((), ()))

        def layer(s_ref, q_ref):
            mu = s_ref[...] * rinv
            var = jnp.maximum(q_ref[...] * rinv - mu * mu, EPS)
            d = jax.lax.dot_general(sel_d, mu, dnm, preferred_element_type=jnp.float32)
            vi = jax.lax.dot_general(sel_i, var, dnm, preferred_element_type=jnp.float32)
            vj = jax.lax.dot_general(sel_j, var, dnm, preferred_element_type=jnp.float32)
            va = 0.5 * (vi + vj) + EPS
            t1 = 0.125 * jnp.sum(d * d / va, axis=1, keepdims=True)
            t2 = 0.25 * jnp.sum(
                jnp.log(va * va / ((vi + EPS) * (vj + EPS))), axis=1, keepdims=True)
            dm = jnp.maximum(t1 + t2, 0.0)
            rho = jnp.exp(-dm)
            return jnp.sum(avec * rho)

        num1 = layer(s1_ref, q1_ref)
        num2 = layer(s2_ref, q2_ref)
        den = jnp.maximum(jnp.sum(avec), EPS)
        total = (num1 + num2) / den
        out_ref[...] = jnp.full((1, 1), jnp.where(layer_valid, total * 0.5, 0.0),
                                jnp.float32)


def kernel(feat_layer1, feat_layer2, logits, y):
    B, D = feat_layer1.shape
    K = logits.shape[1]
    bB = B // N_STEPS
    bL = B // LG_STEPS

    y2 = y.astype(jnp.int32).reshape(B, 1)

    out = pl.pallas_call(
        _fused,
        grid=(N_STEPS,),
        in_specs=[
            pl.BlockSpec((bB, D), lambda i: (i, 0)),
            pl.BlockSpec((bB, D), lambda i: (i, 0)),
            pl.BlockSpec((bL, K), lambda i: (jnp.minimum(i, LG_STEPS - 1), 0)),
            pl.BlockSpec((B, 1), lambda i: (0, 0)),
        ],
        out_specs=pl.BlockSpec((1, 1), lambda i: (0, 0)),
        out_shape=jax.ShapeDtypeStruct((1, 1), jnp.float32),
        scratch_shapes=[
            pltpu.VMEM((1, KPAD), jnp.float32),
            pltpu.VMEM((KPAD, 1), jnp.float32),
            pltpu.VMEM((KPAD, D), jnp.float32),
            pltpu.VMEM((KPAD, D), jnp.float32),
            pltpu.VMEM((KPAD, D), jnp.float32),
            pltpu.VMEM((KPAD, D), jnp.float32),
            pltpu.VMEM((KPAD, K), jnp.float32),
            pltpu.VMEM((KPAD, KPAD), jnp.float32),
            pltpu.VMEM((TOP_M, KPAD), jnp.float32),
            pltpu.VMEM((TOP_M, KPAD), jnp.float32),
            pltpu.VMEM((TOP_M, 1), jnp.float32),
        ],
    )(feat_layer1.astype(jnp.float32), feat_layer2.astype(jnp.float32),
      logits.astype(jnp.float32), y2)
    return out.reshape(())


# speedup vs baseline: 1.1331x; 1.1331x over previous
import jax
import jax.numpy as jnp
from jax import lax
from jax.experimental import pallas as pl
from jax.experimental.pallas import tpu as pltpu

EPS = 1e-06
TOP_M = 64
KPAD = 128
N_STEPS = 16
LG_STEPS = 4
SEL_START = 5
SEL_PER_STEP = 8
BIG_I32 = 2 ** 30


def _lane_allreduce(x, op):
    for sh in (1, 2, 4, 8, 16, 32, 64):
        x = op(x, pltpu.roll(x, sh, 1))
    return x


def _onehot_t(y_ref, n):
    yl = y_ref[...].reshape(1, n)
    ks = jax.lax.broadcasted_iota(jnp.int32, (KPAD, n), 0)
    return (yl == ks).astype(jnp.float32)


def _fused(f1_ref, f2_ref, lg_ref, y_ref, ylg_ref, out_ref,
           cc_ref, s1_ref, q1_ref, s2_ref, q2_ref, p_ref,
           amat_ref, seli_ref, selj_ref, avec_ref):
    step = pl.program_id(0)
    bB = f1_ref.shape[0]
    bL = lg_ref.shape[0]
    K = lg_ref.shape[1]
    dnm = (((1,), (0,)), ((), ()))

    @pl.when(step == 0)
    def _init():
        cc_ref[...] = jnp.zeros_like(cc_ref)
        s1_ref[...] = jnp.zeros_like(s1_ref)
        q1_ref[...] = jnp.zeros_like(q1_ref)
        s2_ref[...] = jnp.zeros_like(s2_ref)
        q2_ref[...] = jnp.zeros_like(q2_ref)
        p_ref[...] = jnp.zeros_like(p_ref)

    z1 = f1_ref[...]
    z2 = f2_ref[...]
    oht = _onehot_t(y_ref, bB)
    s1_ref[...] += jax.lax.dot_general(oht, z1, dnm, preferred_element_type=jnp.float32)
    q1_ref[...] += jax.lax.dot_general(oht, z1 * z1, dnm, preferred_element_type=jnp.float32)
    s2_ref[...] += jax.lax.dot_general(oht, z2, dnm, preferred_element_type=jnp.float32)
    q2_ref[...] += jax.lax.dot_general(oht, z2 * z2, dnm, preferred_element_type=jnp.float32)

    @pl.when(step < LG_STEPS)
    def _logits():
        oht4 = _onehot_t(ylg_ref, bL)
        cc_ref[...] += jax.lax.dot_general(
            oht4, jnp.ones((bL, 1), jnp.float32), dnm,
            preferred_element_type=jnp.float32)
        lg = lg_ref[...]
        m = jnp.max(lg, axis=1, keepdims=True)
        e = jnp.exp(lg - m)
        p = e / jnp.sum(e, axis=1, keepdims=True)
        p_ref[...] += jax.lax.dot_general(oht4, p, dnm,
                                          preferred_element_type=jnp.float32)

    @pl.when(step == LG_STEPS)
    def _build_amat():
        c_col = cc_ref[...]
        rinv = 1.0 / jnp.maximum(c_col, 1.0)
        valid_col = (c_col >= 2.0)
        ri = jax.lax.broadcasted_iota(jnp.int32, (KPAD, KPAD), 0)
        cj = jax.lax.broadcasted_iota(jnp.int32, (KPAD, KPAD), 1)
        eye_kk = (ri == cj).astype(jnp.float32)
        c_lane = jnp.sum(eye_kk * c_col, axis=0, keepdims=True)
        valid_row = (c_lane >= 2.0)
        mean_p = p_ref[...] * rinv
        ei = jax.lax.broadcasted_iota(jnp.int32, (K, KPAD), 0)
        ej = jax.lax.broadcasted_iota(jnp.int32, (K, KPAD), 1)
        pad_eye = (ei == ej).astype(jnp.float32)
        mp = jax.lax.dot_general(mean_p, pad_eye, dnm,
                                 preferred_element_type=jnp.float32)
        alpha = 0.5 * (mp + mp.T)
        keep = jnp.logical_and(cj > ri, jnp.logical_and(valid_col, valid_row))
        amat_ref[...] = jnp.where(keep, alpha, 0.0)

    ri = jax.lax.broadcasted_iota(jnp.int32, (KPAD, KPAD), 0)
    cj = jax.lax.broadcasted_iota(jnp.int32, (KPAD, KPAD), 1)
    fidx = ri * KPAD + cj
    fidxT = cj * KPAD + ri

    for s in range(SEL_START, SEL_START + TOP_M // SEL_PER_STEP):
        @pl.when(step == s)
        def _select(s=s):
            amat = amat_ref[...]
            for it in range(SEL_PER_STEP):
                g = (s - SEL_START) * SEL_PER_STEP + it
                colmax = jnp.max(amat, axis=0, keepdims=True)
                allmax = _lane_allreduce(colmax, jnp.maximum)
                cand = jnp.where(amat == allmax, fidx, BIG_I32)
                colmin = jnp.min(cand, axis=0, keepdims=True)
                allmin = _lane_allreduce(colmin, jnp.minimum)
                msel = fidx == allmin
                jind = jnp.sum(msel.astype(jnp.float32), axis=0, keepdims=True)
                iind = jnp.sum((fidxT == allmin).astype(jnp.float32), axis=0,
                               keepdims=True)
                seli_ref[pl.ds(g, 1), :] = iind
                selj_ref[pl.ds(g, 1), :] = jind
                avec_ref[pl.ds(g, 1), :] = allmax[:, :1]
                amat = jnp.where(msel, -1.0, amat)
            amat_ref[...] = amat

    @pl.when(step == N_STEPS - 1)
    def _finale():
        c_col = cc_ref[...]
        rinv = 1.0 / jnp.maximum(c_col, 1.0)
        valid_col = (c_col >= 2.0)
        num_valid = jnp.sum(valid_col.astype(jnp.float32))
        kept = jnp.sum(jnp.where(valid_col, c_col, 0.0))
        layer_valid = jnp.logical_and(num_valid >= 2.0, kept >= 4.0)

        sel_i = seli_ref[...]
        sel_j = selj_ref[...]
        sel_d = sel_i - sel_j
        avec = avec_ref[...]

        def layer(s_ref, q_ref):
            mu = s_ref[...] * rinv
            var = jnp.maximum(q_ref[...] * rinv - mu * mu, EPS)
            d = jax.lax.dot_general(sel_d, mu, dnm, preferred_element_type=jnp.float32)
            vi = jax.lax.dot_general(sel_i, var, dnm, preferred_element_type=jnp.float32)
            vj = jax.lax.dot_general(sel_j, var, dnm, preferred_element_type=jnp.float32)
            va = 0.5 * (vi + vj) + EPS
            t1 = 0.125 * jnp.sum(d * d / va, axis=1, keepdims=True)
            t2 = 0.25 * jnp.sum(
                jnp.log(va * va / ((vi + EPS) * (vj + EPS))), axis=1, keepdims=True)
            dm = jnp.maximum(t1 + t2, 0.0)
            rho = jnp.exp(-dm)
            return jnp.sum(avec * rho)

        num1 = layer(s1_ref, q1_ref)
        num2 = layer(s2_ref, q2_ref)
        den = jnp.maximum(jnp.sum(avec), EPS)
        total = (num1 + num2) / den
        out_ref[...] = jnp.full((1, 1), jnp.where(layer_valid, total * 0.5, 0.0),
                                jnp.float32)


def kernel(feat_layer1, feat_layer2, logits, y):
    B, D = feat_layer1.shape
    K = logits.shape[1]
    bB = B // N_STEPS
    bL = B // LG_STEPS

    yi = y.astype(jnp.int32)
    y3 = yi.reshape(N_STEPS, 1, bB)
    y3lg = yi.reshape(LG_STEPS, 1, bL)

    out = pl.pallas_call(
        _fused,
        grid=(N_STEPS,),
        in_specs=[
            pl.BlockSpec((bB, D), lambda i: (i, 0)),
            pl.BlockSpec((bB, D), lambda i: (i, 0)),
            pl.BlockSpec((bL, K), lambda i: (jnp.minimum(i, LG_STEPS - 1), 0)),
            pl.BlockSpec((1, 1, bB), lambda i: (i, 0, 0)),
            pl.BlockSpec((1, 1, bL), lambda i: (jnp.minimum(i, LG_STEPS - 1), 0, 0)),
        ],
        out_specs=pl.BlockSpec((1, 1), lambda i: (0, 0)),
        out_shape=jax.ShapeDtypeStruct((1, 1), jnp.float32),
        scratch_shapes=[
            pltpu.VMEM((KPAD, 1), jnp.float32),
            pltpu.VMEM((KPAD, D), jnp.float32),
            pltpu.VMEM((KPAD, D), jnp.float32),
            pltpu.VMEM((KPAD, D), jnp.float32),
            pltpu.VMEM((KPAD, D), jnp.float32),
            pltpu.VMEM((KPAD, K), jnp.float32),
            pltpu.VMEM((KPAD, KPAD), jnp.float32),
            pltpu.VMEM((TOP_M, KPAD), jnp.float32),
            pltpu.VMEM((TOP_M, KPAD), jnp.float32),
            pltpu.VMEM((TOP_M, 1), jnp.float32),
        ],
    )(feat_layer1.astype(jnp.float32), feat_layer2.astype(jnp.float32),
      logits.astype(jnp.float32), y3, y3lg)
    return out.reshape(())
